# baseline (speedup 1.0000x reference)
import functools

import jax
import jax.numpy as jnp
from jax.experimental import pallas as pl

NUM_MAIN_SEQ_ATOMS = 12
NUM_NEIGHBOUR = 32
EMBED_DIM = 32
EPS = 1e-05
BIG = 1e6
KILL = 3.0e38


_SORT8 = [(0, 1), (2, 3), (4, 5), (6, 7),
          (0, 2), (1, 3), (4, 6), (5, 7), (1, 2), (5, 6),
          (0, 4), (1, 5), (2, 6), (3, 7), (2, 4), (3, 5),
          (1, 2), (3, 4), (5, 6)]
_G = 8


def _knn_body(ct_ref, cr_ref, mc_ref, mr_ref, vals_ref, idx_ref):
    kx = ct_ref[0, 0:1, :]
    ky = ct_ref[0, 1:2, :]
    kz = ct_ref[0, 2:3, :]
    qx = cr_ref[0, :, 0:1]
    qy = cr_ref[0, :, 1:2]
    qz = cr_ref[0, :, 2:3]
    dx = kx - qx
    dy = ky - qy
    dz = kz - qz
    d2 = dx * dx + dy * dy
    d2 = d2 + dz * dz
    dist = jnp.sqrt(d2 + 1e-6)
    m2d = mc_ref[0, 0:1, :] * mr_ref[0, :, 0:1]
    dist = dist * m2d + (1.0 - m2d) * BIG

    R, N = dist.shape
    W = N // _G
    colw = jax.lax.broadcasted_iota(jnp.int32, (R, W), 1)
    pad_q = mr_ref[0, :, 0:1] == 0.0

    vs = [dist[:, g * W:(g + 1) * W] for g in range(_G)]
    ix = [colw + g * W for g in range(_G)]
    for (a, b) in _SORT8:
        va, vb, ia, ib = vs[a], vs[b], ix[a], ix[b]
        sw = (va > vb) | ((va == vb) & (ia > ib))
        vs[a] = jnp.where(sw, vb, va)
        vs[b] = jnp.where(sw, va, vb)
        ix[a] = jnp.where(sw, ib, ia)
        ix[b] = jnp.where(sw, ia, ib)

    BIGI = jnp.int32(2 * N)
    for t in range(NUM_NEIGHBOUR):
        y = vs[0]
        m = jnp.min(y, axis=1, keepdims=True)
        cand = jnp.where(y == m, ix[0], BIGI)
        i0 = jnp.min(cand, axis=1, keepdims=True)
        vals_ref[0, :, t:t + 1] = jnp.where(pad_q, BIG, m)
        idx_ref[0, :, t:t + 1] = jnp.where(pad_q, 0, i0)
        if t + 1 < NUM_NEIGHBOUR:
            sel = cand == i0
            for g in range(_G - 1):
                vs[g] = jnp.where(sel, vs[g + 1], vs[g])
                ix[g] = jnp.where(sel, ix[g + 1], ix[g])
            vs[_G - 1] = jnp.where(sel, KILL, vs[_G - 1])


def _encode_body(emb_ref, mr_ref, sc_ref, sh_ref, out_ref):
    m = mr_ref[0]
    feat = emb_ref[...] * m
    masked = feat * m
    cnt = jnp.sum(m, axis=0, keepdims=True)
    cnt = jnp.where(cnt == 0.0, 1.0, cnt)
    mean = jnp.sum(masked, axis=0, keepdims=True) / cnt
    var = jnp.sum((masked - mean) ** 2, axis=0, keepdims=True) / cnt
    std = jnp.sqrt(var + EPS)
    normalized = (feat - mean) / std * sc_ref[...] + sh_ref[...]
    out_ref[0] = normalized * m


def kernel(coords, mask, emb_table, scale, shift):
    B, L, A, _ = coords.shape
    N = L * A
    D = emb_table.shape[-1]
    K = NUM_NEIGHBOUR

    atom_coords = coords.reshape(B, N, 3)
    atom_mask = jnp.broadcast_to(mask[:, :, None], (B, L, A)).reshape(B, N)

    coords_t = atom_coords.transpose(0, 2, 1)
    mask_c = atom_mask[:, None, :]
    mask_r = atom_mask[:, :, None]

    R = 8
    grid = (B, N // R)
    vals, idx = pl.pallas_call(
        _knn_body,
        grid=grid,
        in_specs=[
            pl.BlockSpec((1, 3, N), lambda b, r: (b, 0, 0)),
            pl.BlockSpec((1, R, 3), lambda b, r: (b, r, 0)),
            pl.BlockSpec((1, 1, N), lambda b, r: (b, 0, 0)),
            pl.BlockSpec((1, R, 1), lambda b, r: (b, r, 0)),
        ],
        out_specs=[
            pl.BlockSpec((1, R, K), lambda b, r: (b, r, 0)),
            pl.BlockSpec((1, R, K), lambda b, r: (b, r, 0)),
        ],
        out_shape=[
            jax.ShapeDtypeStruct((B, N, K), jnp.float32),
            jax.ShapeDtypeStruct((B, N, K), jnp.int32),
        ],
    )(coords_t, atom_coords, mask_c, mask_r)

    emb_full = jnp.tile(emb_table, (L, 1))
    encode = pl.pallas_call(
        _encode_body,
        grid=(B,),
        in_specs=[
            pl.BlockSpec((N, D), lambda b: (0, 0)),
            pl.BlockSpec((1, N, 1), lambda b: (b, 0, 0)),
            pl.BlockSpec((1, D), lambda b: (0, 0)),
            pl.BlockSpec((1, D), lambda b: (0, 0)),
        ],
        out_specs=pl.BlockSpec((1, N, D), lambda b: (b, 0, 0)),
        out_shape=jax.ShapeDtypeStruct((B, N, D), jnp.float32),
    )(emb_full, mask_r, scale.reshape(1, D), shift.reshape(1, D))

    return (atom_coords, atom_mask, encode, vals, idx)


# speedup vs baseline: 8.4996x; 8.4996x over previous
import functools

import jax
import jax.numpy as jnp
from jax.experimental import pallas as pl

NUM_MAIN_SEQ_ATOMS = 12
NUM_NEIGHBOUR = 32
EMBED_DIM = 32
EPS = 1e-05
BIG = 1e6
KILL = 3.0e38


_SORT8 = [(0, 1), (2, 3), (4, 5), (6, 7),
          (0, 2), (1, 3), (4, 6), (5, 7), (1, 2), (5, 6),
          (0, 4), (1, 5), (2, 6), (3, 7), (2, 4), (3, 5),
          (1, 2), (3, 4), (5, 6)]
_G = 8


def _knn_body(ct_ref, cr_ref, mc_ref, mr_ref, vals_ref, idx_ref):
    kx = ct_ref[0, 0:1, :]
    ky = ct_ref[0, 1:2, :]
    kz = ct_ref[0, 2:3, :]
    qx = cr_ref[0, :, 0:1]
    qy = cr_ref[0, :, 1:2]
    qz = cr_ref[0, :, 2:3]
    dx = kx - qx
    dy = ky - qy
    dz = kz - qz
    d2 = dx * dx + dy * dy
    d2 = d2 + dz * dz
    dist = jnp.sqrt(d2 + 1e-6)
    m2d = mc_ref[0, 0:1, :] * mr_ref[0, :, 0:1]
    dist = dist * m2d + (1.0 - m2d) * BIG

    R, N = dist.shape
    W = N // _G
    colw = jax.lax.broadcasted_iota(jnp.int32, (R, W), 1)
    pad_q = mr_ref[0, :, 0:1] == 0.0

    vs = [dist[:, g * W:(g + 1) * W] for g in range(_G)]
    ix = [colw + g * W for g in range(_G)]
    for (a, b) in _SORT8:
        va, vb, ia, ib = vs[a], vs[b], ix[a], ix[b]
        sw = (va > vb) | ((va == vb) & (ia > ib))
        vs[a] = jnp.where(sw, vb, va)
        vs[b] = jnp.where(sw, va, vb)
        ix[a] = jnp.where(sw, ib, ia)
        ix[b] = jnp.where(sw, ia, ib)

    BIGI = jnp.int32(2 * N)
    for t in range(NUM_NEIGHBOUR):
        y = vs[0]
        m = jnp.min(y, axis=1, keepdims=True)
        cand = jnp.where(y == m, ix[0], BIGI)
        i0 = jnp.min(cand, axis=1, keepdims=True)
        vals_ref[0, :, t:t + 1] = jnp.where(pad_q, BIG, m)
        idx_ref[0, :, t:t + 1] = jnp.where(pad_q, 0, i0)
        if t + 1 < NUM_NEIGHBOUR:
            sel = cand == i0
            for g in range(_G - 1):
                vs[g] = jnp.where(sel, vs[g + 1], vs[g])
                ix[g] = jnp.where(sel, ix[g + 1], ix[g])
            vs[_G - 1] = jnp.where(sel, KILL, vs[_G - 1])


def _encode_body(emb_ref, mr_ref, sc_ref, sh_ref, out_ref):
    m = mr_ref[0]
    feat = emb_ref[...] * m
    masked = feat * m
    cnt = jnp.sum(m, axis=0, keepdims=True)
    cnt = jnp.where(cnt == 0.0, 1.0, cnt)
    mean = jnp.sum(masked, axis=0, keepdims=True) / cnt
    var = jnp.sum((masked - mean) ** 2, axis=0, keepdims=True) / cnt
    std = jnp.sqrt(var + EPS)
    normalized = (feat - mean) / std * sc_ref[...] + sh_ref[...]
    out_ref[0] = normalized * m


def kernel(coords, mask, emb_table, scale, shift):
    B, L, A, _ = coords.shape
    N = L * A
    D = emb_table.shape[-1]
    K = NUM_NEIGHBOUR

    atom_coords = coords.reshape(B, N, 3)
    atom_mask = jnp.broadcast_to(mask[:, :, None], (B, L, A)).reshape(B, N)

    coords_t = atom_coords.transpose(0, 2, 1)
    mask_c = atom_mask[:, None, :]
    mask_r = atom_mask[:, :, None]

    R = 256
    grid = (B, N // R)
    vals, idx = pl.pallas_call(
        _knn_body,
        grid=grid,
        in_specs=[
            pl.BlockSpec((1, 3, N), lambda b, r: (b, 0, 0)),
            pl.BlockSpec((1, R, 3), lambda b, r: (b, r, 0)),
            pl.BlockSpec((1, 1, N), lambda b, r: (b, 0, 0)),
            pl.BlockSpec((1, R, 1), lambda b, r: (b, r, 0)),
        ],
        out_specs=[
            pl.BlockSpec((1, R, K), lambda b, r: (b, r, 0)),
            pl.BlockSpec((1, R, K), lambda b, r: (b, r, 0)),
        ],
        out_shape=[
            jax.ShapeDtypeStruct((B, N, K), jnp.float32),
            jax.ShapeDtypeStruct((B, N, K), jnp.int32),
        ],
    )(coords_t, atom_coords, mask_c, mask_r)

    emb_full = jnp.tile(emb_table, (L, 1))
    encode = pl.pallas_call(
        _encode_body,
        grid=(B,),
        in_specs=[
            pl.BlockSpec((N, D), lambda b: (0, 0)),
            pl.BlockSpec((1, N, 1), lambda b: (b, 0, 0)),
            pl.BlockSpec((1, D), lambda b: (0, 0)),
            pl.BlockSpec((1, D), lambda b: (0, 0)),
        ],
        out_specs=pl.BlockSpec((1, N, D), lambda b: (b, 0, 0)),
        out_shape=jax.ShapeDtypeStruct((B, N, D), jnp.float32),
    )(emb_full, mask_r, scale.reshape(1, D), shift.reshape(1, D))

    return (atom_coords, atom_mask, encode, vals, idx)
